# baseline (speedup 1.0000x reference)
import functools

import jax
import jax.numpy as jnp
from jax import lax
from jax.experimental import pallas as pl
from jax.experimental.pallas import tpu as pltpu
from jax.experimental.pallas import tpu_sc as plsc

NC = 2
NS = 16
NW = NC * NS
GW = 128
NBUF = 6


def _silu(v):
    return v / (1.0 + jnp.exp(-v))


def _pre_body(x_ref, posp_ref, w1a_ref, b1_ref, w1b_ref, wn1a_ref, bn1_ref,
              xap_ref, xbp_ref, xn_ref):
    x = x_ref[...]
    posp = posp_ref[...]
    z = jnp.zeros((x.shape[0], GW - 80), jnp.float32)
    xa = jnp.dot(x, w1a_ref[...], preferred_element_type=jnp.float32) + b1_ref[...]
    xb = jnp.dot(x, w1b_ref[...], preferred_element_type=jnp.float32)
    xap_ref[...] = jnp.concatenate([xa, posp, z], axis=1)
    xbp_ref[...] = jnp.concatenate([xb, -posp, z], axis=1)
    xn_ref[...] = jnp.dot(x, wn1a_ref[...], preferred_element_type=jnp.float32) + bn1_ref[...]


def _edge_body(ga_ref, gb_ref, eat_ref, ones_ref, w1c_ref, w1d_ref, w2_ref,
               b2_ref, wc1_ref, bc1_ref, wc2_ref, bc2_ref, md_ref):
    g = ga_ref[...] + gb_ref[...]
    diffp = g[:, 64:80]
    sq = jnp.dot(diffp * diffp, ones_ref[...], preferred_element_type=jnp.float32)
    dist = jnp.sqrt(sq)
    eterm = lax.dot_general(eat_ref[...], w1d_ref[...],
                            (((0,), (0,)), ((), ())),
                            preferred_element_type=jnp.float32)
    pre = g[:, :64] + dist * w1c_ref[...] + eterm
    msg = _silu(jnp.dot(_silu(pre), w2_ref[...], preferred_element_type=jnp.float32)
                + b2_ref[...])
    c1 = _silu(jnp.dot(msg, wc1_ref[...], preferred_element_type=jnp.float32)
               + bc1_ref[...])
    cw = jnp.dot(c1, wc2_ref[...], preferred_element_type=jnp.float32) + bc2_ref[...]
    z = jnp.zeros((g.shape[0], GW - 80), jnp.float32)
    md_ref[...] = jnp.concatenate([msg, diffp * cw, z], axis=1)


def _node_body(acc0_ref, acc1_ref, acc2_ref, acc3_ref, xn_ref, posp_ref,
               wn1b_ref, wn2_ref, bn2_ref, xnew_ref, posn_ref):
    acc = ((acc0_ref[...] + acc1_ref[...])
           + (acc2_ref[...] + acc3_ref[...]))
    h = _silu(xn_ref[...] + jnp.dot(acc[:, :64], wn1b_ref[...],
                                    preferred_element_type=jnp.float32))
    xnew_ref[...] = jnp.dot(h, wn2_ref[...], preferred_element_type=jnp.float32) + bn2_ref[...]
    posn_ref[...] = posp_ref[...] + acc[:, 64:80]


def _make_gather(E, N, CHUNK):
    epw = E // NW
    nch = epw // CHUNK
    nmain = (nch - 1) // NBUF * NBUF
    mesh = plsc.VectorSubcoreMesh(core_axis_name="c", subcore_axis_name="s",
                                  num_cores=NC, num_subcores=NS)

    @functools.partial(
        pl.kernel, mesh=mesh,
        out_type=[jax.ShapeDtypeStruct((E, GW), jnp.float32),
                  jax.ShapeDtypeStruct((E, GW), jnp.float32)],
        scratch_types=[pltpu.VMEM((epw,), jnp.int32),
                       pltpu.VMEM((epw,), jnp.int32)]
                      + [pltpu.VMEM((CHUNK, GW), jnp.float32)] * (2 * NBUF)
                      + [pltpu.SemaphoreType.DMA] * (4 * NBUF),
    )
    def gather_k(xap_hbm, xbp_hbm, row_hbm, col_hbm, ga_hbm, gb_hbm,
                 idx_r, idx_c, *bufs_sems):
        buf_a = bufs_sems[0:NBUF]
        buf_b = bufs_sems[NBUF:2 * NBUF]
        sem_a = bufs_sems[2 * NBUF:3 * NBUF]
        sem_b = bufs_sems[3 * NBUF:4 * NBUF]
        ssem_a = bufs_sems[4 * NBUF:5 * NBUF]
        ssem_b = bufs_sems[5 * NBUF:6 * NBUF]
        wid = lax.axis_index("s") * NC + lax.axis_index("c")
        base = wid * epw
        pltpu.sync_copy(row_hbm.at[pl.ds(base, epw)], idx_r)
        pltpu.sync_copy(col_hbm.at[pl.ds(base, epw)], idx_c)

        def fire(i, b):
            pltpu.async_copy(xap_hbm.at[idx_r.at[pl.ds(i * CHUNK, CHUNK)]],
                             buf_a[b], sem_a[b])
            pltpu.async_copy(xbp_hbm.at[idx_c.at[pl.ds(i * CHUNK, CHUNK)]],
                             buf_b[b], sem_b[b])

        def drain_store(i, b):
            off = base + i * CHUNK
            pltpu.make_async_copy(xap_hbm.at[idx_r.at[pl.ds(0, CHUNK)]],
                                  buf_a[b], sem_a[b]).wait()
            pltpu.make_async_copy(xbp_hbm.at[idx_c.at[pl.ds(0, CHUNK)]],
                                  buf_b[b], sem_b[b]).wait()
            pltpu.async_copy(buf_a[b], ga_hbm.at[pl.ds(off, CHUNK)], ssem_a[b])
            pltpu.async_copy(buf_b[b], gb_hbm.at[pl.ds(off, CHUNK)], ssem_b[b])

        def wait_stores(b):
            pltpu.make_async_copy(buf_a[b], ga_hbm.at[pl.ds(0, CHUNK)],
                                  ssem_a[b]).wait()
            pltpu.make_async_copy(buf_b[b], gb_hbm.at[pl.ds(0, CHUNK)],
                                  ssem_b[b]).wait()

        for b in range(NBUF):
            fire(b, b)

        def body(j, carry):
            for b in range(NBUF):
                i = j * NBUF + b
                drain_store(i, b)

                @pl.when(i + NBUF < nch)
                def _():
                    wait_stores(b)
                    fire(i + NBUF, b)
            return carry

        lax.fori_loop(0, nmain // NBUF, body, 0)
        for i in range(nmain, nch):
            drain_store(i, i % NBUF)
        for b in range(NBUF):
            wait_stores(b)

    return gather_k


def _make_scatter(E, N, CHUNK):
    epw = E // NW
    nch = epw // CHUNK
    nmain = (nch - 1) // NBUF * NBUF
    npc = -(-N // NS) // 8 * 8
    npc_last = N - npc * (NS - 1)
    assert npc_last > 0
    mesh = plsc.VectorSubcoreMesh(core_axis_name="c", subcore_axis_name="s",
                                  num_cores=NC, num_subcores=NS)

    @functools.partial(
        pl.kernel, mesh=mesh,
        out_type=jax.ShapeDtypeStruct((NC, N, GW), jnp.float32),
        scratch_types=[pltpu.VMEM_SHARED((N, GW), jnp.float32)]
                      + [pltpu.VMEM((CHUNK,), jnp.int32)] * NBUF
                      + [pltpu.VMEM((CHUNK, GW), jnp.float32)] * NBUF
                      + [pltpu.SemaphoreType.DMA] * (2 * NBUF),
    )
    def scatter_k(md_hbm, row_hbm, zeros_hbm, acc_hbm, acc_sh, *bufs_sems):
        idxs = bufs_sems[0:NBUF]
        bufs = bufs_sems[NBUF:2 * NBUF]
        isems = bufs_sems[2 * NBUF:3 * NBUF]
        dsems = bufs_sems[3 * NBUF:4 * NBUF]
        cid = lax.axis_index("c")
        sid = lax.axis_index("s")
        wid = sid * NC + cid
        base = wid * epw

        def fire(i, b):
            pltpu.async_copy(row_hbm.at[pl.ds(base + i * CHUNK, CHUNK)],
                             idxs[b], isems[b])
            pltpu.async_copy(md_hbm.at[pl.ds(base + i * CHUNK, CHUNK)],
                             bufs[b], dsems[b])

        def drain_scatter(i, b):
            pltpu.make_async_copy(row_hbm.at[pl.ds(0, CHUNK)], idxs[b],
                                  isems[b]).wait()
            pltpu.make_async_copy(md_hbm.at[pl.ds(0, CHUNK)], bufs[b],
                                  dsems[b]).wait()
            pltpu.sync_copy(bufs[b], acc_sh.at[idxs[b]], add=True)

        @pl.when(sid < NS - 1)
        def _():
            pltpu.sync_copy(zeros_hbm.at[pl.ds(sid * npc, npc)],
                            acc_sh.at[pl.ds(sid * npc, npc)])

        @pl.when(sid == NS - 1)
        def _():
            pltpu.sync_copy(zeros_hbm.at[pl.ds((NS - 1) * npc, npc_last)],
                            acc_sh.at[pl.ds((NS - 1) * npc, npc_last)])

        for b in range(NBUF):
            fire(b, b)
        plsc.subcore_barrier()

        def body(j, carry):
            for b in range(NBUF):
                i = j * NBUF + b
                drain_scatter(i, b)

                @pl.when(i + NBUF < nch)
                def _():
                    fire(i + NBUF, b)
            return carry

        lax.fori_loop(0, nmain // NBUF, body, 0)
        for i in range(nmain, nch):
            drain_scatter(i, i % NBUF)
        plsc.subcore_barrier()

        @pl.when(sid < NS - 1)
        def _():
            pltpu.sync_copy(acc_sh.at[pl.ds(sid * npc, npc)],
                            acc_hbm.at[cid, pl.ds(sid * npc, npc)])

        @pl.when(sid == NS - 1)
        def _():
            pltpu.sync_copy(acc_sh.at[pl.ds((NS - 1) * npc, npc_last)],
                            acc_hbm.at[cid, pl.ds((NS - 1) * npc, npc_last)])

    return scatter_k


def kernel(x, pos, edge_index, edge_attr, W1, b1, W2, b2,
           Wn1, bn1, Wn2, bn2, Wc1, bc1, Wc2, bc2):
    N, D = x.shape
    E = edge_index.shape[1]
    H = W2.shape[0]
    assert D == 128 and H == 64
    assert N % NS == 0

    row = edge_index[0]
    col = edge_index[1]
    posp = jnp.pad(pos, ((0, 0), (0, 16 - pos.shape[1])))
    w1a = W1[:D]
    w1b = W1[D:2 * D]
    w1c = W1[2 * D:2 * D + 1]
    w1d = W1[2 * D + 1:]
    wn1a = Wn1[:D]
    wn1b = Wn1[D:]
    wc2t = jnp.tile(Wc2, (1, 16))
    bc2t = jnp.broadcast_to(bc2.reshape(1, 1), (1, 16))

    bpre = 2000
    xap, xbp, xn = pl.pallas_call(
        _pre_body,
        grid=(N // bpre,),
        in_specs=[
            pl.BlockSpec((bpre, D), lambda i: (i, 0)),
            pl.BlockSpec((bpre, 16), lambda i: (i, 0)),
            pl.BlockSpec((D, H), lambda i: (0, 0)),
            pl.BlockSpec((1, H), lambda i: (0, 0)),
            pl.BlockSpec((D, H), lambda i: (0, 0)),
            pl.BlockSpec((D, H), lambda i: (0, 0)),
            pl.BlockSpec((1, H), lambda i: (0, 0)),
        ],
        out_specs=[
            pl.BlockSpec((bpre, GW), lambda i: (i, 0)),
            pl.BlockSpec((bpre, GW), lambda i: (i, 0)),
            pl.BlockSpec((bpre, H), lambda i: (i, 0)),
        ],
        out_shape=[
            jax.ShapeDtypeStruct((N, GW), jnp.float32),
            jax.ShapeDtypeStruct((N, GW), jnp.float32),
            jax.ShapeDtypeStruct((N, H), jnp.float32),
        ],
    )(x, posp, w1a, b1.reshape(1, H), w1b, wn1a, bn1.reshape(1, H))

    eat = edge_attr.T
    zeros = jnp.zeros((N, GW), jnp.float32)
    bedge = 2560
    splits = [(0, 192000, 40), (192000, 128000, 40)]
    assert sum(sz for _, sz, _ in splits) == E
    accs = []
    for start, EH, chunk in splits:
        row_h = lax.slice_in_dim(row, start, start + EH)
        col_h = lax.slice_in_dim(col, start, start + EH)
        ga, gb = _make_gather(EH, N, chunk)(xap, xbp, row_h, col_h)
        md = pl.pallas_call(
            _edge_body,
            grid=(EH // bedge,),
            in_specs=[
                pl.BlockSpec((bedge, GW), lambda i: (i, 0)),
                pl.BlockSpec((bedge, GW), lambda i: (i, 0)),
                pl.BlockSpec((16, bedge),
                             lambda i, s=start // bedge: (0, i + s)),
                pl.BlockSpec((16, H), lambda i: (0, 0)),
                pl.BlockSpec((1, H), lambda i: (0, 0)),
                pl.BlockSpec((16, H), lambda i: (0, 0)),
                pl.BlockSpec((H, H), lambda i: (0, 0)),
                pl.BlockSpec((1, H), lambda i: (0, 0)),
                pl.BlockSpec((H, H), lambda i: (0, 0)),
                pl.BlockSpec((1, H), lambda i: (0, 0)),
                pl.BlockSpec((H, 16), lambda i: (0, 0)),
                pl.BlockSpec((1, 16), lambda i: (0, 0)),
            ],
            out_specs=pl.BlockSpec((bedge, GW), lambda i: (i, 0)),
            out_shape=jax.ShapeDtypeStruct((EH, GW), jnp.float32),
        )(ga, gb, eat, jnp.ones((16, H), jnp.float32), w1c, w1d, W2,
          b2.reshape(1, H), Wc1, bc1.reshape(1, H), wc2t, bc2t)
        acc_pair = _make_scatter(EH, N, chunk)(md, row_h, zeros)
        accs.extend([acc_pair[0], acc_pair[1]])

    bnode = 2000
    x_new, posn = pl.pallas_call(
        _node_body,
        grid=(N // bnode,),
        in_specs=[
            pl.BlockSpec((bnode, GW), lambda i: (i, 0)),
            pl.BlockSpec((bnode, GW), lambda i: (i, 0)),
            pl.BlockSpec((bnode, GW), lambda i: (i, 0)),
            pl.BlockSpec((bnode, GW), lambda i: (i, 0)),
            pl.BlockSpec((bnode, H), lambda i: (i, 0)),
            pl.BlockSpec((bnode, 16), lambda i: (i, 0)),
            pl.BlockSpec((H, H), lambda i: (0, 0)),
            pl.BlockSpec((H, D), lambda i: (0, 0)),
            pl.BlockSpec((1, D), lambda i: (0, 0)),
        ],
        out_specs=[
            pl.BlockSpec((bnode, D), lambda i: (i, 0)),
            pl.BlockSpec((bnode, 16), lambda i: (i, 0)),
        ],
        out_shape=[
            jax.ShapeDtypeStruct((N, D), jnp.float32),
            jax.ShapeDtypeStruct((N, 16), jnp.float32),
        ],
    )(accs[0], accs[1], accs[2], accs[3], xn, posp, wn1b, Wn2,
      bn2.reshape(1, D))

    return (x_new, posn[:, :3])


# speedup vs baseline: 1.0241x; 1.0241x over previous
import functools

import jax
import jax.numpy as jnp
from jax import lax
from jax.experimental import pallas as pl
from jax.experimental.pallas import tpu as pltpu
from jax.experimental.pallas import tpu_sc as plsc

NC = 2
NS = 16
NW = NC * NS
GW = 128
NBUF = 6


def _silu(v):
    return v / (1.0 + jnp.exp(-v))


def _pre_body(x_ref, posp_ref, w1a_ref, b1_ref, w1b_ref, wn1a_ref, bn1_ref,
              xap_ref, xbp_ref, xn_ref):
    x = x_ref[...]
    posp = posp_ref[...]
    z = jnp.zeros((x.shape[0], GW - 80), jnp.float32)
    xa = jnp.dot(x, w1a_ref[...], preferred_element_type=jnp.float32) + b1_ref[...]
    xb = jnp.dot(x, w1b_ref[...], preferred_element_type=jnp.float32)
    xap_ref[...] = jnp.concatenate([xa, posp, z], axis=1)
    xbp_ref[...] = jnp.concatenate([xb, -posp, z], axis=1)
    xn_ref[...] = jnp.dot(x, wn1a_ref[...], preferred_element_type=jnp.float32) + bn1_ref[...]


def _edge_body(ga_ref, gb_ref, eat_ref, ones_ref, w1c_ref, w1d_ref, w2_ref,
               b2_ref, wc1_ref, bc1_ref, wc2_ref, bc2_ref, md_ref):
    g = ga_ref[...] + gb_ref[...]
    diffp = g[:, 64:80]
    sq = jnp.dot(diffp * diffp, ones_ref[...], preferred_element_type=jnp.float32)
    dist = jnp.sqrt(sq)
    eterm = lax.dot_general(eat_ref[...], w1d_ref[...],
                            (((0,), (0,)), ((), ())),
                            preferred_element_type=jnp.float32)
    pre = g[:, :64] + dist * w1c_ref[...] + eterm
    msg = _silu(jnp.dot(_silu(pre), w2_ref[...], preferred_element_type=jnp.float32)
                + b2_ref[...])
    c1 = _silu(jnp.dot(msg, wc1_ref[...], preferred_element_type=jnp.float32)
               + bc1_ref[...])
    cw = jnp.dot(c1, wc2_ref[...], preferred_element_type=jnp.float32) + bc2_ref[...]
    z = jnp.zeros((g.shape[0], GW - 80), jnp.float32)
    md_ref[...] = jnp.concatenate([msg, diffp * cw, z], axis=1)


def _node_body(acc0_ref, acc1_ref, acc2_ref, acc3_ref, xn_ref, posp_ref,
               wn1b_ref, wn2_ref, bn2_ref, xnew_ref, posn_ref):
    acc = ((acc0_ref[...] + acc1_ref[...])
           + (acc2_ref[...] + acc3_ref[...]))
    h = _silu(xn_ref[...] + jnp.dot(acc[:, :64], wn1b_ref[...],
                                    preferred_element_type=jnp.float32))
    xnew_ref[...] = jnp.dot(h, wn2_ref[...], preferred_element_type=jnp.float32) + bn2_ref[...]
    posn_ref[...] = posp_ref[...] + acc[:, 64:80]


def _make_gather(E, N, CHUNK):
    epw = E // NW
    nch = epw // CHUNK
    nmain = (nch - 1) // NBUF * NBUF
    mesh = plsc.VectorSubcoreMesh(core_axis_name="c", subcore_axis_name="s",
                                  num_cores=NC, num_subcores=NS)

    @functools.partial(
        pl.kernel, mesh=mesh,
        out_type=[jax.ShapeDtypeStruct((E, GW), jnp.float32),
                  jax.ShapeDtypeStruct((E, GW), jnp.float32)],
        scratch_types=[pltpu.VMEM((epw,), jnp.int32),
                       pltpu.VMEM((epw,), jnp.int32)]
                      + [pltpu.VMEM((CHUNK, GW), jnp.float32)] * (2 * NBUF)
                      + [pltpu.SemaphoreType.DMA] * (4 * NBUF),
    )
    def gather_k(xap_hbm, xbp_hbm, row_hbm, col_hbm, ga_hbm, gb_hbm,
                 idx_r, idx_c, *bufs_sems):
        buf_a = bufs_sems[0:NBUF]
        buf_b = bufs_sems[NBUF:2 * NBUF]
        sem_a = bufs_sems[2 * NBUF:3 * NBUF]
        sem_b = bufs_sems[3 * NBUF:4 * NBUF]
        ssem_a = bufs_sems[4 * NBUF:5 * NBUF]
        ssem_b = bufs_sems[5 * NBUF:6 * NBUF]
        wid = lax.axis_index("s") * NC + lax.axis_index("c")
        base = wid * epw
        pltpu.sync_copy(row_hbm.at[pl.ds(base, epw)], idx_r)
        pltpu.sync_copy(col_hbm.at[pl.ds(base, epw)], idx_c)

        def fire(i, b):
            pltpu.async_copy(xap_hbm.at[idx_r.at[pl.ds(i * CHUNK, CHUNK)]],
                             buf_a[b], sem_a[b])
            pltpu.async_copy(xbp_hbm.at[idx_c.at[pl.ds(i * CHUNK, CHUNK)]],
                             buf_b[b], sem_b[b])

        def drain_store(i, b):
            off = base + i * CHUNK
            pltpu.make_async_copy(xap_hbm.at[idx_r.at[pl.ds(0, CHUNK)]],
                                  buf_a[b], sem_a[b]).wait()
            pltpu.make_async_copy(xbp_hbm.at[idx_c.at[pl.ds(0, CHUNK)]],
                                  buf_b[b], sem_b[b]).wait()
            pltpu.async_copy(buf_a[b], ga_hbm.at[pl.ds(off, CHUNK)], ssem_a[b])
            pltpu.async_copy(buf_b[b], gb_hbm.at[pl.ds(off, CHUNK)], ssem_b[b])

        def wait_stores(b):
            pltpu.make_async_copy(buf_a[b], ga_hbm.at[pl.ds(0, CHUNK)],
                                  ssem_a[b]).wait()
            pltpu.make_async_copy(buf_b[b], gb_hbm.at[pl.ds(0, CHUNK)],
                                  ssem_b[b]).wait()

        for b in range(NBUF):
            fire(b, b)

        def body(j, carry):
            for b in range(NBUF):
                i = j * NBUF + b
                drain_store(i, b)

                @pl.when(i + NBUF < nch)
                def _():
                    wait_stores(b)
                    fire(i + NBUF, b)
            return carry

        lax.fori_loop(0, nmain // NBUF, body, 0)
        for i in range(nmain, nch):
            drain_store(i, i % NBUF)
        for b in range(NBUF):
            wait_stores(b)

    return gather_k


def _make_scatter(E, N, CHUNK):
    epw = E // NW
    nch = epw // CHUNK
    nmain = (nch - 1) // NBUF * NBUF
    npc = -(-N // NS) // 8 * 8
    npc_last = N - npc * (NS - 1)
    assert npc_last > 0
    mesh = plsc.VectorSubcoreMesh(core_axis_name="c", subcore_axis_name="s",
                                  num_cores=NC, num_subcores=NS)

    @functools.partial(
        pl.kernel, mesh=mesh,
        out_type=jax.ShapeDtypeStruct((NC, N, GW), jnp.float32),
        scratch_types=[pltpu.VMEM_SHARED((N, GW), jnp.float32)]
                      + [pltpu.VMEM((CHUNK,), jnp.int32)] * NBUF
                      + [pltpu.VMEM((CHUNK, GW), jnp.float32)] * NBUF
                      + [pltpu.SemaphoreType.DMA] * (2 * NBUF),
    )
    def scatter_k(md_hbm, row_hbm, zeros_hbm, acc_hbm, acc_sh, *bufs_sems):
        idxs = bufs_sems[0:NBUF]
        bufs = bufs_sems[NBUF:2 * NBUF]
        isems = bufs_sems[2 * NBUF:3 * NBUF]
        dsems = bufs_sems[3 * NBUF:4 * NBUF]
        cid = lax.axis_index("c")
        sid = lax.axis_index("s")
        wid = sid * NC + cid
        base = wid * epw

        def fire(i, b):
            pltpu.async_copy(row_hbm.at[pl.ds(base + i * CHUNK, CHUNK)],
                             idxs[b], isems[b])
            pltpu.async_copy(md_hbm.at[pl.ds(base + i * CHUNK, CHUNK)],
                             bufs[b], dsems[b])

        def drain_scatter(i, b):
            pltpu.make_async_copy(row_hbm.at[pl.ds(0, CHUNK)], idxs[b],
                                  isems[b]).wait()
            pltpu.make_async_copy(md_hbm.at[pl.ds(0, CHUNK)], bufs[b],
                                  dsems[b]).wait()
            pltpu.sync_copy(bufs[b], acc_sh.at[idxs[b]], add=True)

        @pl.when(sid < NS - 1)
        def _():
            pltpu.sync_copy(zeros_hbm.at[pl.ds(sid * npc, npc)],
                            acc_sh.at[pl.ds(sid * npc, npc)])

        @pl.when(sid == NS - 1)
        def _():
            pltpu.sync_copy(zeros_hbm.at[pl.ds((NS - 1) * npc, npc_last)],
                            acc_sh.at[pl.ds((NS - 1) * npc, npc_last)])

        for b in range(NBUF):
            fire(b, b)
        plsc.subcore_barrier()

        def body(j, carry):
            for b in range(NBUF):
                i = j * NBUF + b
                drain_scatter(i, b)

                @pl.when(i + NBUF < nch)
                def _():
                    fire(i + NBUF, b)
            return carry

        lax.fori_loop(0, nmain // NBUF, body, 0)
        for i in range(nmain, nch):
            drain_scatter(i, i % NBUF)
        plsc.subcore_barrier()

        @pl.when(sid < NS - 1)
        def _():
            pltpu.sync_copy(acc_sh.at[pl.ds(sid * npc, npc)],
                            acc_hbm.at[cid, pl.ds(sid * npc, npc)])

        @pl.when(sid == NS - 1)
        def _():
            pltpu.sync_copy(acc_sh.at[pl.ds((NS - 1) * npc, npc_last)],
                            acc_hbm.at[cid, pl.ds((NS - 1) * npc, npc_last)])

    return scatter_k


def kernel(x, pos, edge_index, edge_attr, W1, b1, W2, b2,
           Wn1, bn1, Wn2, bn2, Wc1, bc1, Wc2, bc2):
    N, D = x.shape
    E = edge_index.shape[1]
    H = W2.shape[0]
    assert D == 128 and H == 64
    assert N % NS == 0

    row = edge_index[0]
    col = edge_index[1]
    posp = jnp.pad(pos, ((0, 0), (0, 16 - pos.shape[1])))
    w1a = W1[:D]
    w1b = W1[D:2 * D]
    w1c = W1[2 * D:2 * D + 1]
    w1d = W1[2 * D + 1:]
    wn1a = Wn1[:D]
    wn1b = Wn1[D:]
    wc2t = jnp.tile(Wc2, (1, 16))
    bc2t = jnp.broadcast_to(bc2.reshape(1, 1), (1, 16))

    bpre = 2000
    xap, xbp, xn = pl.pallas_call(
        _pre_body,
        grid=(N // bpre,),
        in_specs=[
            pl.BlockSpec((bpre, D), lambda i: (i, 0)),
            pl.BlockSpec((bpre, 16), lambda i: (i, 0)),
            pl.BlockSpec((D, H), lambda i: (0, 0)),
            pl.BlockSpec((1, H), lambda i: (0, 0)),
            pl.BlockSpec((D, H), lambda i: (0, 0)),
            pl.BlockSpec((D, H), lambda i: (0, 0)),
            pl.BlockSpec((1, H), lambda i: (0, 0)),
        ],
        out_specs=[
            pl.BlockSpec((bpre, GW), lambda i: (i, 0)),
            pl.BlockSpec((bpre, GW), lambda i: (i, 0)),
            pl.BlockSpec((bpre, H), lambda i: (i, 0)),
        ],
        out_shape=[
            jax.ShapeDtypeStruct((N, GW), jnp.float32),
            jax.ShapeDtypeStruct((N, GW), jnp.float32),
            jax.ShapeDtypeStruct((N, H), jnp.float32),
        ],
    )(x, posp, w1a, b1.reshape(1, H), w1b, wn1a, bn1.reshape(1, H))

    eat = edge_attr.T
    zeros = jnp.zeros((N, GW), jnp.float32)
    bedge = 3200
    splits = [(0, 160000, 40), (160000, 160000, 40)]
    assert sum(sz for _, sz, _ in splits) == E
    assert all(sz % bedge == 0 and sz % (NW * ch) == 0 for _, sz, ch in splits)
    accs = []
    for start, EH, chunk in splits:
        row_h = lax.slice_in_dim(row, start, start + EH)
        col_h = lax.slice_in_dim(col, start, start + EH)
        ga, gb = _make_gather(EH, N, chunk)(xap, xbp, row_h, col_h)
        md = pl.pallas_call(
            _edge_body,
            grid=(EH // bedge,),
            in_specs=[
                pl.BlockSpec((bedge, GW), lambda i: (i, 0)),
                pl.BlockSpec((bedge, GW), lambda i: (i, 0)),
                pl.BlockSpec((16, bedge),
                             lambda i, s=start // bedge: (0, i + s)),
                pl.BlockSpec((16, H), lambda i: (0, 0)),
                pl.BlockSpec((1, H), lambda i: (0, 0)),
                pl.BlockSpec((16, H), lambda i: (0, 0)),
                pl.BlockSpec((H, H), lambda i: (0, 0)),
                pl.BlockSpec((1, H), lambda i: (0, 0)),
                pl.BlockSpec((H, H), lambda i: (0, 0)),
                pl.BlockSpec((1, H), lambda i: (0, 0)),
                pl.BlockSpec((H, 16), lambda i: (0, 0)),
                pl.BlockSpec((1, 16), lambda i: (0, 0)),
            ],
            out_specs=pl.BlockSpec((bedge, GW), lambda i: (i, 0)),
            out_shape=jax.ShapeDtypeStruct((EH, GW), jnp.float32),
        )(ga, gb, eat, jnp.ones((16, H), jnp.float32), w1c, w1d, W2,
          b2.reshape(1, H), Wc1, bc1.reshape(1, H), wc2t, bc2t)
        acc_pair = _make_scatter(EH, N, chunk)(md, row_h, zeros)
        accs.extend([acc_pair[0], acc_pair[1]])

    bnode = 2000
    x_new, posn = pl.pallas_call(
        _node_body,
        grid=(N // bnode,),
        in_specs=[
            pl.BlockSpec((bnode, GW), lambda i: (i, 0)),
            pl.BlockSpec((bnode, GW), lambda i: (i, 0)),
            pl.BlockSpec((bnode, GW), lambda i: (i, 0)),
            pl.BlockSpec((bnode, GW), lambda i: (i, 0)),
            pl.BlockSpec((bnode, H), lambda i: (i, 0)),
            pl.BlockSpec((bnode, 16), lambda i: (i, 0)),
            pl.BlockSpec((H, H), lambda i: (0, 0)),
            pl.BlockSpec((H, D), lambda i: (0, 0)),
            pl.BlockSpec((1, D), lambda i: (0, 0)),
        ],
        out_specs=[
            pl.BlockSpec((bnode, D), lambda i: (i, 0)),
            pl.BlockSpec((bnode, 16), lambda i: (i, 0)),
        ],
        out_shape=[
            jax.ShapeDtypeStruct((N, D), jnp.float32),
            jax.ShapeDtypeStruct((N, 16), jnp.float32),
        ],
    )(accs[0], accs[1], accs[2], accs[3], xn, posp, wn1b, Wn2,
      bn2.reshape(1, D))

    return (x_new, posn[:, :3])


# speedup vs baseline: 1.0552x; 1.0304x over previous
import functools

import jax
import jax.numpy as jnp
from jax import lax
from jax.experimental import pallas as pl
from jax.experimental.pallas import tpu as pltpu
from jax.experimental.pallas import tpu_sc as plsc

NC = 2
NS = 16
NW = NC * NS
GW = 128
NBUF = 6


def _silu(v):
    return v / (1.0 + jnp.exp(-v))


def _pre_body(x_ref, posp_ref, w1a_ref, b1_ref, w1b_ref, wn1a_ref, bn1_ref,
              xap_ref, xbp_ref, xn_ref):
    x = x_ref[...]
    posp = posp_ref[...]
    z = jnp.zeros((x.shape[0], GW - 80), jnp.float32)
    xa = jnp.dot(x, w1a_ref[...], preferred_element_type=jnp.float32) + b1_ref[...]
    xb = jnp.dot(x, w1b_ref[...], preferred_element_type=jnp.float32)
    xap_ref[...] = jnp.concatenate([xa, posp, z], axis=1)
    xbp_ref[...] = jnp.concatenate([xb, -posp, z], axis=1)
    xn_ref[...] = jnp.dot(x, wn1a_ref[...], preferred_element_type=jnp.float32) + bn1_ref[...]


def _edge_body(ga_ref, gb_ref, eat_ref, ones_ref, w1c_ref, w1d_ref, w2_ref,
               b2_ref, wc1_ref, bc1_ref, wc2_ref, bc2_ref, md_ref):
    g = ga_ref[...] + gb_ref[...]
    diffp = g[:, 64:80]
    sq = jnp.dot(diffp * diffp, ones_ref[...], preferred_element_type=jnp.float32)
    dist = jnp.sqrt(sq)
    eterm = lax.dot_general(eat_ref[...], w1d_ref[...],
                            (((0,), (0,)), ((), ())),
                            preferred_element_type=jnp.float32)
    pre = g[:, :64] + dist * w1c_ref[...] + eterm
    msg = _silu(jnp.dot(_silu(pre), w2_ref[...], preferred_element_type=jnp.float32)
                + b2_ref[...])
    c1 = _silu(jnp.dot(msg, wc1_ref[...], preferred_element_type=jnp.float32)
               + bc1_ref[...])
    cw = jnp.dot(c1, wc2_ref[...], preferred_element_type=jnp.float32) + bc2_ref[...]
    z = jnp.zeros((g.shape[0], GW - 80), jnp.float32)
    md_ref[...] = jnp.concatenate([msg, diffp * cw, z], axis=1)


def _node_body(acc0_ref, acc1_ref, acc2_ref, acc3_ref, acc4_ref, acc5_ref,
               xn_ref, posp_ref, wn1b_ref, wn2_ref, bn2_ref,
               xnew_ref, posn_ref):
    acc = ((acc0_ref[...] + acc1_ref[...])
           + (acc2_ref[...] + acc3_ref[...])
           + (acc4_ref[...] + acc5_ref[...]))
    h = _silu(xn_ref[...] + jnp.dot(acc[:, :64], wn1b_ref[...],
                                    preferred_element_type=jnp.float32))
    xnew_ref[...] = jnp.dot(h, wn2_ref[...], preferred_element_type=jnp.float32) + bn2_ref[...]
    posn_ref[...] = posp_ref[...] + acc[:, 64:80]


def _make_gather(E, N, CHUNK):
    epw = E // NW
    nch = epw // CHUNK
    nmain = (nch - 1) // NBUF * NBUF
    mesh = plsc.VectorSubcoreMesh(core_axis_name="c", subcore_axis_name="s",
                                  num_cores=NC, num_subcores=NS)

    @functools.partial(
        pl.kernel, mesh=mesh,
        out_type=[jax.ShapeDtypeStruct((E, GW), jnp.float32),
                  jax.ShapeDtypeStruct((E, GW), jnp.float32)],
        scratch_types=[pltpu.VMEM((epw,), jnp.int32),
                       pltpu.VMEM((epw,), jnp.int32)]
                      + [pltpu.VMEM((CHUNK, GW), jnp.float32)] * (2 * NBUF)
                      + [pltpu.SemaphoreType.DMA] * (4 * NBUF),
    )
    def gather_k(xap_hbm, xbp_hbm, row_hbm, col_hbm, ga_hbm, gb_hbm,
                 idx_r, idx_c, *bufs_sems):
        buf_a = bufs_sems[0:NBUF]
        buf_b = bufs_sems[NBUF:2 * NBUF]
        sem_a = bufs_sems[2 * NBUF:3 * NBUF]
        sem_b = bufs_sems[3 * NBUF:4 * NBUF]
        ssem_a = bufs_sems[4 * NBUF:5 * NBUF]
        ssem_b = bufs_sems[5 * NBUF:6 * NBUF]
        wid = lax.axis_index("s") * NC + lax.axis_index("c")
        base = wid * epw
        pltpu.sync_copy(row_hbm.at[pl.ds(base, epw)], idx_r)
        pltpu.sync_copy(col_hbm.at[pl.ds(base, epw)], idx_c)

        def fire(i, b):
            pltpu.async_copy(xap_hbm.at[idx_r.at[pl.ds(i * CHUNK, CHUNK)]],
                             buf_a[b], sem_a[b])
            pltpu.async_copy(xbp_hbm.at[idx_c.at[pl.ds(i * CHUNK, CHUNK)]],
                             buf_b[b], sem_b[b])

        def drain_store(i, b):
            off = base + i * CHUNK
            pltpu.make_async_copy(xap_hbm.at[idx_r.at[pl.ds(0, CHUNK)]],
                                  buf_a[b], sem_a[b]).wait()
            pltpu.make_async_copy(xbp_hbm.at[idx_c.at[pl.ds(0, CHUNK)]],
                                  buf_b[b], sem_b[b]).wait()
            pltpu.async_copy(buf_a[b], ga_hbm.at[pl.ds(off, CHUNK)], ssem_a[b])
            pltpu.async_copy(buf_b[b], gb_hbm.at[pl.ds(off, CHUNK)], ssem_b[b])

        def wait_stores(b):
            pltpu.make_async_copy(buf_a[b], ga_hbm.at[pl.ds(0, CHUNK)],
                                  ssem_a[b]).wait()
            pltpu.make_async_copy(buf_b[b], gb_hbm.at[pl.ds(0, CHUNK)],
                                  ssem_b[b]).wait()

        for b in range(NBUF):
            fire(b, b)

        def body(j, carry):
            for b in range(NBUF):
                i = j * NBUF + b
                drain_store(i, b)

                @pl.when(i + NBUF < nch)
                def _():
                    wait_stores(b)
                    fire(i + NBUF, b)
            return carry

        lax.fori_loop(0, nmain // NBUF, body, 0)
        for i in range(nmain, nch):
            drain_store(i, i % NBUF)
        for b in range(NBUF):
            wait_stores(b)

    return gather_k


def _make_scatter(E, N, CHUNK):
    epw = E // NW
    nch = epw // CHUNK
    nmain = (nch - 1) // NBUF * NBUF
    npc = -(-N // NS) // 8 * 8
    npc_last = N - npc * (NS - 1)
    assert npc_last > 0
    mesh = plsc.VectorSubcoreMesh(core_axis_name="c", subcore_axis_name="s",
                                  num_cores=NC, num_subcores=NS)

    @functools.partial(
        pl.kernel, mesh=mesh,
        out_type=jax.ShapeDtypeStruct((NC, N, GW), jnp.float32),
        scratch_types=[pltpu.VMEM_SHARED((N, GW), jnp.float32)]
                      + [pltpu.VMEM((CHUNK,), jnp.int32)] * NBUF
                      + [pltpu.VMEM((CHUNK, GW), jnp.float32)] * NBUF
                      + [pltpu.SemaphoreType.DMA] * (2 * NBUF),
    )
    def scatter_k(md_hbm, row_hbm, zeros_hbm, acc_hbm, acc_sh, *bufs_sems):
        idxs = bufs_sems[0:NBUF]
        bufs = bufs_sems[NBUF:2 * NBUF]
        isems = bufs_sems[2 * NBUF:3 * NBUF]
        dsems = bufs_sems[3 * NBUF:4 * NBUF]
        cid = lax.axis_index("c")
        sid = lax.axis_index("s")
        wid = sid * NC + cid
        base = wid * epw

        def fire(i, b):
            pltpu.async_copy(row_hbm.at[pl.ds(base + i * CHUNK, CHUNK)],
                             idxs[b], isems[b])
            pltpu.async_copy(md_hbm.at[pl.ds(base + i * CHUNK, CHUNK)],
                             bufs[b], dsems[b])

        def drain_scatter(i, b):
            pltpu.make_async_copy(row_hbm.at[pl.ds(0, CHUNK)], idxs[b],
                                  isems[b]).wait()
            pltpu.make_async_copy(md_hbm.at[pl.ds(0, CHUNK)], bufs[b],
                                  dsems[b]).wait()
            pltpu.sync_copy(bufs[b], acc_sh.at[idxs[b]], add=True)

        @pl.when(sid < NS - 1)
        def _():
            pltpu.sync_copy(zeros_hbm.at[pl.ds(sid * npc, npc)],
                            acc_sh.at[pl.ds(sid * npc, npc)])

        @pl.when(sid == NS - 1)
        def _():
            pltpu.sync_copy(zeros_hbm.at[pl.ds((NS - 1) * npc, npc_last)],
                            acc_sh.at[pl.ds((NS - 1) * npc, npc_last)])

        for b in range(NBUF):
            fire(b, b)
        plsc.subcore_barrier()

        def body(j, carry):
            for b in range(NBUF):
                i = j * NBUF + b
                drain_scatter(i, b)

                @pl.when(i + NBUF < nch)
                def _():
                    fire(i + NBUF, b)
            return carry

        lax.fori_loop(0, nmain // NBUF, body, 0)
        for i in range(nmain, nch):
            drain_scatter(i, i % NBUF)
        plsc.subcore_barrier()

        @pl.when(sid < NS - 1)
        def _():
            pltpu.sync_copy(acc_sh.at[pl.ds(sid * npc, npc)],
                            acc_hbm.at[cid, pl.ds(sid * npc, npc)])

        @pl.when(sid == NS - 1)
        def _():
            pltpu.sync_copy(acc_sh.at[pl.ds((NS - 1) * npc, npc_last)],
                            acc_hbm.at[cid, pl.ds((NS - 1) * npc, npc_last)])

    return scatter_k


def kernel(x, pos, edge_index, edge_attr, W1, b1, W2, b2,
           Wn1, bn1, Wn2, bn2, Wc1, bc1, Wc2, bc2):
    N, D = x.shape
    E = edge_index.shape[1]
    H = W2.shape[0]
    assert D == 128 and H == 64
    assert N % NS == 0

    row = edge_index[0]
    col = edge_index[1]
    posp = jnp.pad(pos, ((0, 0), (0, 16 - pos.shape[1])))
    w1a = W1[:D]
    w1b = W1[D:2 * D]
    w1c = W1[2 * D:2 * D + 1]
    w1d = W1[2 * D + 1:]
    wn1a = Wn1[:D]
    wn1b = Wn1[D:]
    wc2t = jnp.tile(Wc2, (1, 16))
    bc2t = jnp.broadcast_to(bc2.reshape(1, 1), (1, 16))

    bpre = 2000
    xap, xbp, xn = pl.pallas_call(
        _pre_body,
        grid=(N // bpre,),
        in_specs=[
            pl.BlockSpec((bpre, D), lambda i: (i, 0)),
            pl.BlockSpec((bpre, 16), lambda i: (i, 0)),
            pl.BlockSpec((D, H), lambda i: (0, 0)),
            pl.BlockSpec((1, H), lambda i: (0, 0)),
            pl.BlockSpec((D, H), lambda i: (0, 0)),
            pl.BlockSpec((D, H), lambda i: (0, 0)),
            pl.BlockSpec((1, H), lambda i: (0, 0)),
        ],
        out_specs=[
            pl.BlockSpec((bpre, GW), lambda i: (i, 0)),
            pl.BlockSpec((bpre, GW), lambda i: (i, 0)),
            pl.BlockSpec((bpre, H), lambda i: (i, 0)),
        ],
        out_shape=[
            jax.ShapeDtypeStruct((N, GW), jnp.float32),
            jax.ShapeDtypeStruct((N, GW), jnp.float32),
            jax.ShapeDtypeStruct((N, H), jnp.float32),
        ],
    )(x, posp, w1a, b1.reshape(1, H), w1b, wn1a, bn1.reshape(1, H))

    eat = edge_attr.T
    zeros = jnp.zeros((N, GW), jnp.float32)
    bedge = 3200
    splits = [(0, 128000, 40), (128000, 128000, 40), (256000, 64000, 40)]
    assert sum(sz for _, sz, _ in splits) == E
    assert all(sz % bedge == 0 and sz % (NW * ch) == 0 for _, sz, ch in splits)
    accs = []
    for start, EH, chunk in splits:
        row_h = lax.slice_in_dim(row, start, start + EH)
        col_h = lax.slice_in_dim(col, start, start + EH)
        ga, gb = _make_gather(EH, N, chunk)(xap, xbp, row_h, col_h)
        md = pl.pallas_call(
            _edge_body,
            grid=(EH // bedge,),
            in_specs=[
                pl.BlockSpec((bedge, GW), lambda i: (i, 0)),
                pl.BlockSpec((bedge, GW), lambda i: (i, 0)),
                pl.BlockSpec((16, bedge),
                             lambda i, s=start // bedge: (0, i + s)),
                pl.BlockSpec((16, H), lambda i: (0, 0)),
                pl.BlockSpec((1, H), lambda i: (0, 0)),
                pl.BlockSpec((16, H), lambda i: (0, 0)),
                pl.BlockSpec((H, H), lambda i: (0, 0)),
                pl.BlockSpec((1, H), lambda i: (0, 0)),
                pl.BlockSpec((H, H), lambda i: (0, 0)),
                pl.BlockSpec((1, H), lambda i: (0, 0)),
                pl.BlockSpec((H, 16), lambda i: (0, 0)),
                pl.BlockSpec((1, 16), lambda i: (0, 0)),
            ],
            out_specs=pl.BlockSpec((bedge, GW), lambda i: (i, 0)),
            out_shape=jax.ShapeDtypeStruct((EH, GW), jnp.float32),
        )(ga, gb, eat, jnp.ones((16, H), jnp.float32), w1c, w1d, W2,
          b2.reshape(1, H), Wc1, bc1.reshape(1, H), wc2t, bc2t)
        acc_pair = _make_scatter(EH, N, chunk)(md, row_h, zeros)
        accs.extend([acc_pair[0], acc_pair[1]])

    bnode = 2000
    x_new, posn = pl.pallas_call(
        _node_body,
        grid=(N // bnode,),
        in_specs=[
            pl.BlockSpec((bnode, GW), lambda i: (i, 0)),
            pl.BlockSpec((bnode, GW), lambda i: (i, 0)),
            pl.BlockSpec((bnode, GW), lambda i: (i, 0)),
            pl.BlockSpec((bnode, GW), lambda i: (i, 0)),
            pl.BlockSpec((bnode, GW), lambda i: (i, 0)),
            pl.BlockSpec((bnode, GW), lambda i: (i, 0)),
            pl.BlockSpec((bnode, H), lambda i: (i, 0)),
            pl.BlockSpec((bnode, 16), lambda i: (i, 0)),
            pl.BlockSpec((H, H), lambda i: (0, 0)),
            pl.BlockSpec((H, D), lambda i: (0, 0)),
            pl.BlockSpec((1, D), lambda i: (0, 0)),
        ],
        out_specs=[
            pl.BlockSpec((bnode, D), lambda i: (i, 0)),
            pl.BlockSpec((bnode, 16), lambda i: (i, 0)),
        ],
        out_shape=[
            jax.ShapeDtypeStruct((N, D), jnp.float32),
            jax.ShapeDtypeStruct((N, 16), jnp.float32),
        ],
    )(*accs, xn, posp, wn1b, Wn2, bn2.reshape(1, D))

    return (x_new, posn[:, :3])
